# Initial kernel scaffold; baseline (speedup 1.0000x reference)
import jax
import jax.numpy as jnp
from jax.experimental import pallas as pl


def kernel(x, edge_index, edge_attr, W1, b1, W2, b2, gamma, beta):
    raise NotImplementedError("write your pallas kernel here")


# speedup vs baseline: 3.1335x; 3.1335x over previous
import functools

import jax
import jax.numpy as jnp
from jax import lax
from jax.experimental import pallas as pl
from jax.experimental.pallas import tpu as pltpu
from jax.experimental.pallas import tpu_sc as plsc

N_NODES = 10000
N_EDGES = 320000
D_NODE = 128
D_EDGE = 16
D_OUT = 128

NC = 2
NS = 16
NW = NC * NS
EPW = N_EDGES // NW
EB = 40
NB = EPW // EB
N_PAD = 10240
ROWS_PER_TILE = N_PAD // NS


def _prep_body(x_ref, w1_ref, p_ref, q_ref):
    w1a = w1_ref[0:D_NODE, :]
    w1b = w1_ref[D_NODE:2 * D_NODE, :]
    x = x_ref[...]
    p_ref[...] = jnp.dot(x, w1a - w1b, preferred_element_type=jnp.float32)
    q_ref[...] = jnp.dot(x, w1b, preferred_element_type=jnp.float32)


def _prep(x, w1):
    return pl.pallas_call(
        _prep_body,
        out_shape=[
            jax.ShapeDtypeStruct((N_NODES, D_NODE), jnp.float32),
            jax.ShapeDtypeStruct((N_NODES, D_NODE), jnp.float32),
        ],
    )(x, w1)


EBLK = 8000


def _edge_bias_body(ea_ref, w1_ref, b1_ref, c_ref):
    w1c = w1_ref[2 * D_NODE:2 * D_NODE + D_EDGE, :]
    c_ref[...] = (
        jnp.dot(ea_ref[...], w1c, preferred_element_type=jnp.float32)
        + b1_ref[...]
    )


def _edge_bias(edge_attr, w1, b1_row):
    return pl.pallas_call(
        _edge_bias_body,
        grid=(N_EDGES // EBLK,),
        in_specs=[
            pl.BlockSpec((EBLK, D_EDGE), lambda i: (i, 0)),
            pl.BlockSpec((2 * D_NODE + D_EDGE, D_OUT), lambda i: (0, 0)),
            pl.BlockSpec((1, D_OUT), lambda i: (0, 0)),
        ],
        out_specs=pl.BlockSpec((EBLK, D_OUT), lambda i: (i, 0)),
        out_shape=jax.ShapeDtypeStruct((N_EDGES, D_OUT), jnp.float32),
    )(edge_attr, w1, b1_row)


def _sc_body(p_hbm, q_hbm, c_hbm, src_hbm, dst_hbm, out_hbm, deg_hbm,
             s_sh, dst_v, src_v, p_v, q_v, c_v, st_v, hist_v,
             sem_p, sem_q, sem_c):
    c = lax.axis_index("c")
    s = lax.axis_index("s")
    wid = s * NC + c

    zvec = jnp.zeros((16,), jnp.float32)
    ones = jnp.ones((16,), jnp.float32)

    def _zero_st(i, _):
        for k in range(D_OUT // 16):
            st_v[i, pl.ds(k * 16, 16)] = zvec
        return 0
    lax.fori_loop(0, EB, _zero_st, 0)
    for j in range(ROWS_PER_TILE // EB):
        pltpu.sync_copy(
            st_v, s_sh.at[pl.ds(s * ROWS_PER_TILE + j * EB, EB)])

    def _zero_hist(i, _):
        hist_v[pl.ds(i * 16, 16)] = zvec
        return 0
    lax.fori_loop(0, N_PAD // 16, _zero_hist, 0)

    plsc.subcore_barrier()

    def _block(g, _):
        base = wid * EPW + g * EB
        pltpu.sync_copy(dst_hbm.at[pl.ds(base, EB)], dst_v)
        pltpu.sync_copy(src_hbm.at[pl.ds(base, EB)], src_v)
        cp_p = pltpu.async_copy(p_hbm.at[dst_v], p_v, sem_p)
        cp_q = pltpu.async_copy(q_hbm.at[src_v], q_v, sem_q)
        cp_c = pltpu.async_copy(c_hbm.at[pl.ds(base, EB)], c_v, sem_c)
        for i in range(EB // 16):
            plsc.addupdate_scatter(hist_v, [dst_v[pl.ds(i * 16, 16)]], ones)
        cp_p.wait()
        cp_q.wait()
        cp_c.wait()

        def _edge(e, _):
            for k in range(D_OUT // 16):
                o = k * 16
                v = (p_v[e, pl.ds(o, 16)] + q_v[e, pl.ds(o, 16)]
                     + c_v[e, pl.ds(o, 16)])
                st_v[e, pl.ds(o, 16)] = jnp.maximum(v, 0.0)
            return 0
        lax.fori_loop(0, EB, _edge, 0)

        pltpu.sync_copy(st_v, s_sh.at[dst_v], add=True)
        return 0

    lax.fori_loop(0, NB, _block, 0)

    plsc.subcore_barrier()

    pltpu.sync_copy(s_sh.at[pl.ds(s * ROWS_PER_TILE, ROWS_PER_TILE)],
                    out_hbm.at[c, pl.ds(s * ROWS_PER_TILE, ROWS_PER_TILE)])
    pltpu.sync_copy(hist_v, deg_hbm.at[c, s])


_sc_edge = pl.kernel(
    _sc_body,
    out_type=(jax.ShapeDtypeStruct((NC, N_PAD, D_OUT), jnp.float32),
              jax.ShapeDtypeStruct((NC, NS, N_PAD), jnp.float32)),
    mesh=plsc.VectorSubcoreMesh(core_axis_name="c", subcore_axis_name="s"),
    compiler_params=pltpu.CompilerParams(needs_layout_passes=False),
    scratch_types=[
        pltpu.VMEM_SHARED((N_PAD, D_OUT), jnp.float32),
        pltpu.VMEM((EB,), jnp.int32),
        pltpu.VMEM((EB,), jnp.int32),
        pltpu.VMEM((EB, D_NODE), jnp.float32),
        pltpu.VMEM((EB, D_NODE), jnp.float32),
        pltpu.VMEM((EB, D_OUT), jnp.float32),
        pltpu.VMEM((EB, D_OUT), jnp.float32),
        pltpu.VMEM((N_PAD,), jnp.float32),
        pltpu.SemaphoreType.DMA,
        pltpu.SemaphoreType.DMA,
        pltpu.SemaphoreType.DMA,
    ],
)


def _final_body(sp_ref, deg_ref, w2_ref, b2_ref, gamma_ref, beta_ref,
                out_ref):
    h = sp_ref[0, :N_NODES, :] + sp_ref[1, :N_NODES, :]
    deg = jnp.sum(deg_ref[...], axis=(0, 1))[:N_NODES].reshape(N_NODES, 1)
    agg = (jnp.dot(h, w2_ref[...], preferred_element_type=jnp.float32)
           + deg * b2_ref[...])
    mean = jnp.mean(agg, axis=0, keepdims=True)
    var = jnp.mean((agg - mean) ** 2, axis=0, keepdims=True)
    out = (agg - mean) * lax.rsqrt(var + 1e-5) * gamma_ref[...] + beta_ref[...]
    out_ref[...] = jnp.maximum(out, 0.0)


def _final(spart, deg, w2, b2_row, gamma_row, beta_row):
    return pl.pallas_call(
        _final_body,
        out_shape=jax.ShapeDtypeStruct((N_NODES, D_OUT), jnp.float32),
    )(spart, deg, w2, b2_row, gamma_row, beta_row)


def kernel(x, edge_index, edge_attr, W1, b1, W2, b2, gamma, beta):
    idx = edge_index.astype(jnp.int32)
    src = idx[0]
    dst = idx[1]
    p, q = _prep(x, W1)
    c = _edge_bias(edge_attr, W1, b1.reshape(1, D_OUT))
    spart, deg = _sc_edge(p, q, c, src, dst)
    return _final(spart, deg, W2, b2.reshape(1, D_OUT),
                  gamma.reshape(1, D_OUT), beta.reshape(1, D_OUT))


# speedup vs baseline: 4.1090x; 1.3113x over previous
import functools

import jax
import jax.numpy as jnp
from jax import lax
from jax.experimental import pallas as pl
from jax.experimental.pallas import tpu as pltpu
from jax.experimental.pallas import tpu_sc as plsc

N_NODES = 10000
N_EDGES = 320000
D_NODE = 128
D_EDGE = 16
D_OUT = 128

NC = 2
NS = 16
NW = NC * NS
EPW = N_EDGES // NW
EB = 40
NB = EPW // EB
N_PAD = 10240
ROWS_PER_TILE = N_PAD // NS


def _prep_body(x_ref, w1_ref, p_ref, q_ref):
    w1a = w1_ref[0:D_NODE, :]
    w1b = w1_ref[D_NODE:2 * D_NODE, :]
    x = x_ref[...]
    p_ref[...] = jnp.dot(x, w1a - w1b, preferred_element_type=jnp.float32)
    q_ref[...] = jnp.dot(x, w1b, preferred_element_type=jnp.float32)


def _prep(x, w1):
    return pl.pallas_call(
        _prep_body,
        out_shape=[
            jax.ShapeDtypeStruct((N_NODES, D_NODE), jnp.float32),
            jax.ShapeDtypeStruct((N_NODES, D_NODE), jnp.float32),
        ],
    )(x, w1)


EBLK = 8000


def _edge_bias_body(ea_ref, w1_ref, b1_ref, c_ref):
    w1c = w1_ref[2 * D_NODE:2 * D_NODE + D_EDGE, :]
    c_ref[...] = (
        jnp.dot(ea_ref[...], w1c, preferred_element_type=jnp.float32)
        + b1_ref[...]
    )


def _edge_bias(edge_attr, w1, b1_row):
    return pl.pallas_call(
        _edge_bias_body,
        grid=(N_EDGES // EBLK,),
        in_specs=[
            pl.BlockSpec((EBLK, D_EDGE), lambda i: (i, 0)),
            pl.BlockSpec((2 * D_NODE + D_EDGE, D_OUT), lambda i: (0, 0)),
            pl.BlockSpec((1, D_OUT), lambda i: (0, 0)),
        ],
        out_specs=pl.BlockSpec((EBLK, D_OUT), lambda i: (i, 0)),
        out_shape=jax.ShapeDtypeStruct((N_EDGES, D_OUT), jnp.float32),
    )(edge_attr, w1, b1_row)


def _sc_body(p_hbm, q_hbm, c_hbm, src_hbm, dst_hbm, out_hbm, deg_hbm,
             s_sh, d0_v, s0_v, d1_v, s1_v, df_v,
             p0_v, q0_v, c0_v, p1_v, q1_v, c1_v, hist_v,
             sem_g0, sem_g1, sem_s0, sem_s1):
    c = lax.axis_index("c")
    s = lax.axis_index("s")
    wid = s * NC + c

    zvec = jnp.zeros((16,), jnp.float32)
    ones = jnp.ones((16,), jnp.float32)

    def _zero_st(i, _):
        for k in range(D_OUT // 16):
            c0_v[i, pl.ds(k * 16, 16)] = zvec
        return 0
    lax.fori_loop(0, EB, _zero_st, 0)
    for j in range(ROWS_PER_TILE // EB):
        pltpu.sync_copy(
            c0_v, s_sh.at[pl.ds(s * ROWS_PER_TILE + j * EB, EB)])

    def _zero_hist(i, _):
        hist_v[pl.ds(i * 16, 16)] = zvec
        return 0
    lax.fori_loop(0, N_PAD // 16, _zero_hist, 0)

    plsc.subcore_barrier()

    def _issue(blk, d_v, s_v, p_v, q_v, cc_v, sem):
        base = blk * EB
        pltpu.sync_copy(dst_hbm.at[pl.ds(base, EB)], d_v)
        pltpu.sync_copy(src_hbm.at[pl.ds(base, EB)], s_v)
        pltpu.async_copy(p_hbm.at[d_v], p_v, sem)
        pltpu.async_copy(q_hbm.at[s_v], q_v, sem)
        pltpu.async_copy(c_hbm.at[pl.ds(base, EB)], cc_v, sem)

    def _drain_gather(d_v, s_v, p_v, q_v, cc_v, sem):
        pltpu.make_async_copy(p_hbm.at[d_v], p_v, sem).wait()
        pltpu.make_async_copy(q_hbm.at[s_v], q_v, sem).wait()
        pltpu.make_async_copy(c_hbm.at[pl.ds(0, EB)], cc_v, sem).wait()

    def _drain_scatter(cc_v, sem):
        pltpu.make_async_copy(cc_v, s_sh.at[pl.ds(0, EB)], sem).wait()

    def _compute(p_v, q_v, cc_v):
        def _edge(e, _):
            for k in range(D_OUT // 16):
                o = k * 16
                v = (p_v[e, pl.ds(o, 16)] + q_v[e, pl.ds(o, 16)]
                     + cc_v[e, pl.ds(o, 16)])
                cc_v[e, pl.ds(o, 16)] = jnp.maximum(v, 0.0)
            return 0
        lax.fori_loop(0, EB, _edge, 0)

    tile_base = wid * NB
    _issue(tile_base, d0_v, s0_v, p0_v, q0_v, c0_v, sem_g0)

    def _outer(g, _):
        b0 = tile_base + 2 * g
        @pl.when(g > 0)
        def _():
            _drain_scatter(c1_v, sem_s1)
        _issue(b0 + 1, d1_v, s1_v, p1_v, q1_v, c1_v, sem_g1)

        pltpu.sync_copy(dst_hbm.at[pl.ds(b0 * EB, 2 * EB)], df_v)
        for i in range(2 * EB // 16):
            plsc.addupdate_scatter(hist_v, [df_v[pl.ds(i * 16, 16)]], ones)

        _drain_gather(d0_v, s0_v, p0_v, q0_v, c0_v, sem_g0)
        _compute(p0_v, q0_v, c0_v)
        pltpu.async_copy(c0_v, s_sh.at[d0_v], sem_s0, add=True)

        @pl.when(g < NB // 2 - 1)
        def _():
            _drain_scatter(c0_v, sem_s0)
            _issue(b0 + 2, d0_v, s0_v, p0_v, q0_v, c0_v, sem_g0)

        _drain_gather(d1_v, s1_v, p1_v, q1_v, c1_v, sem_g1)
        _compute(p1_v, q1_v, c1_v)
        pltpu.async_copy(c1_v, s_sh.at[d1_v], sem_s1, add=True)
        return 0

    lax.fori_loop(0, NB // 2, _outer, 0)
    _drain_scatter(c0_v, sem_s0)
    _drain_scatter(c1_v, sem_s1)

    plsc.subcore_barrier()

    pltpu.sync_copy(s_sh.at[pl.ds(s * ROWS_PER_TILE, ROWS_PER_TILE)],
                    out_hbm.at[c, pl.ds(s * ROWS_PER_TILE, ROWS_PER_TILE)])
    pltpu.sync_copy(hist_v, deg_hbm.at[c, s])


_sc_edge = pl.kernel(
    _sc_body,
    out_type=(jax.ShapeDtypeStruct((NC, N_PAD, D_OUT), jnp.float32),
              jax.ShapeDtypeStruct((NC, NS, N_PAD), jnp.float32)),
    mesh=plsc.VectorSubcoreMesh(core_axis_name="c", subcore_axis_name="s"),
    compiler_params=pltpu.CompilerParams(needs_layout_passes=False),
    scratch_types=[
        pltpu.VMEM_SHARED((N_PAD, D_OUT), jnp.float32),
        pltpu.VMEM((EB,), jnp.int32),
        pltpu.VMEM((EB,), jnp.int32),
        pltpu.VMEM((EB,), jnp.int32),
        pltpu.VMEM((EB,), jnp.int32),
        pltpu.VMEM((2 * EB,), jnp.int32),
        pltpu.VMEM((EB, D_NODE), jnp.float32),
        pltpu.VMEM((EB, D_NODE), jnp.float32),
        pltpu.VMEM((EB, D_OUT), jnp.float32),
        pltpu.VMEM((EB, D_NODE), jnp.float32),
        pltpu.VMEM((EB, D_NODE), jnp.float32),
        pltpu.VMEM((EB, D_OUT), jnp.float32),
        pltpu.VMEM((N_PAD,), jnp.float32),
        pltpu.SemaphoreType.DMA,
        pltpu.SemaphoreType.DMA,
        pltpu.SemaphoreType.DMA,
        pltpu.SemaphoreType.DMA,
    ],
)


def _final_body(sp_ref, deg_ref, w2_ref, b2_ref, gamma_ref, beta_ref,
                out_ref):
    h = sp_ref[0, :N_NODES, :] + sp_ref[1, :N_NODES, :]
    deg = jnp.sum(deg_ref[...], axis=(0, 1))[:N_NODES].reshape(N_NODES, 1)
    agg = (jnp.dot(h, w2_ref[...], preferred_element_type=jnp.float32)
           + deg * b2_ref[...])
    mean = jnp.mean(agg, axis=0, keepdims=True)
    var = jnp.mean((agg - mean) ** 2, axis=0, keepdims=True)
    out = (agg - mean) * lax.rsqrt(var + 1e-5) * gamma_ref[...] + beta_ref[...]
    out_ref[...] = jnp.maximum(out, 0.0)


def _final(spart, deg, w2, b2_row, gamma_row, beta_row):
    return pl.pallas_call(
        _final_body,
        out_shape=jax.ShapeDtypeStruct((N_NODES, D_OUT), jnp.float32),
    )(spart, deg, w2, b2_row, gamma_row, beta_row)


def kernel(x, edge_index, edge_attr, W1, b1, W2, b2, gamma, beta):
    idx = edge_index.astype(jnp.int32)
    src = idx[0]
    dst = idx[1]
    p, q = _prep(x, W1)
    c = _edge_bias(edge_attr, W1, b1.reshape(1, D_OUT))
    spart, deg = _sc_edge(p, q, c, src, dst)
    return _final(spart, deg, W2, b2.reshape(1, D_OUT),
                  gamma.reshape(1, D_OUT), beta.reshape(1, D_OUT))
